# baseline (speedup 1.0000x reference)
import jax
import jax.numpy as jnp
from jax import lax
from jax.experimental import pallas as pl
from jax.experimental.pallas import tpu as pltpu
from jax.experimental.pallas import tpu_sc as plsc

_LANES = 16
_WORKERS = 32


def _make_sc_body(num_heads, s_k, q_per_worker, k_starts, win):
    def body(table_hbm, idx_hbm, out_hbm, table_v, idx_v, out_v):
        wid = lax.axis_index("s") * 2 + lax.axis_index("c")
        row = table_v.shape[0] // num_heads
        q0s = wid * (q_per_worker * s_k)
        a0 = pl.multiple_of(
            lax.shift_left(lax.shift_right_logical(q0s, 7), 7), 128
        )
        shift = q0s - a0
        pltpu.sync_copy(table_hbm, table_v)
        pltpu.sync_copy(idx_hbm.at[pl.ds(a0, win)], idx_v)

        @plsc.parallel_loop(0, q_per_worker)
        def q_loop(qi):
            idx_base = shift + qi * s_k
            for k0 in k_starts:
                iv = idx_v[pl.ds(idx_base + k0, _LANES)]
                for hh in range(num_heads):
                    out_v[qi * num_heads + hh, pl.ds(k0, _LANES)] = (
                        plsc.load_gather(table_v, [iv + hh * row])
                    )
        t0 = wid * (q_per_worker * num_heads)
        pltpu.sync_copy(
            out_v, out_hbm.at[pl.ds(t0, q_per_worker * num_heads), :]
        )

    return body


def _sc_gather(table_flat, idx_flat, num_heads, s_k, q_per_worker, k_starts, win):
    n_rows = _WORKERS * q_per_worker * num_heads
    mesh = plsc.VectorSubcoreMesh(core_axis_name="c", subcore_axis_name="s")
    return pl.kernel(
        _make_sc_body(num_heads, s_k, q_per_worker, k_starts, win),
        out_type=jax.ShapeDtypeStruct((n_rows, s_k), jnp.float32),
        mesh=mesh,
        compiler_params=pltpu.CompilerParams(
            needs_layout_passes=False, use_tc_tiling_on_sc=True
        ),
        scratch_types=[
            pltpu.VMEM((table_flat.shape[0],), jnp.float32),
            pltpu.VMEM((win,), jnp.int32),
            pltpu.VMEM((q_per_worker * num_heads, s_k), jnp.float32),
        ],
    )(table_flat, idx_flat)


def _add_body(x_ref, pos_ref, o_ref):
    o_ref[...] = x_ref[...] + pos_ref[...]


def kernel(inputs, relative_position_bias_table, relative_position_index):
    b, h, s_q, s_k = inputs.shape

    n_tasks = s_q * h
    q_per_worker = -(-s_q // _WORKERS)

    k_starts = list(range(0, s_k - _LANES + 1, _LANES))
    if k_starts[-1] + _LANES < s_k:
        k_starts.append(s_k - _LANES)

    span = 128 + (q_per_worker - 1) * s_k + k_starts[-1] + _LANES
    win = ((span + 127) // 128) * 128
    a0_max = ((_WORKERS - 1) * q_per_worker * s_k) >> 7 << 7
    idx_len = a0_max + win

    idx = relative_position_index[:s_q, :s_k].astype(jnp.int32).reshape(-1)
    idx_pad = jnp.pad(idx, (0, idx_len - idx.size))
    nrd = relative_position_bias_table.shape[1]
    row = ((nrd + 127) // 128) * 128
    table_flat = jnp.pad(
        relative_position_bias_table, ((0, 0), (0, row - nrd))
    ).reshape(-1)

    pos2d = _sc_gather(
        table_flat, idx_pad, h, s_k, q_per_worker, k_starts, win
    )
    pos = pos2d[:n_tasks].reshape(s_q, h, s_k)

    xt = jnp.transpose(inputs, (0, 2, 1, 3))
    out = pl.pallas_call(
        _add_body,
        out_shape=jax.ShapeDtypeStruct((b, s_q, h, s_k), jnp.float32),
        grid=(b // 4,),
        in_specs=[
            pl.BlockSpec((4, s_q, h, s_k), lambda i: (i, 0, 0, 0)),
            pl.BlockSpec((s_q, h, s_k), lambda i: (0, 0, 0)),
        ],
        out_specs=pl.BlockSpec((4, s_q, h, s_k), lambda i: (i, 0, 0, 0)),
    )(xt, pos)
    return jnp.transpose(out, (0, 2, 1, 3))


# speedup vs baseline: 1.0968x; 1.0968x over previous
import jax
import jax.numpy as jnp
from jax import lax
from jax.experimental import pallas as pl
from jax.experimental.pallas import tpu as pltpu
from jax.experimental.pallas import tpu_sc as plsc

_LANES = 16
_WORKERS = 32


def _make_sc_body(num_heads, s_k, q_per_worker, k_starts, win):
    def body(table_hbm, idx_hbm, out_hbm, table_v, idx_v, out_v):
        wid = lax.axis_index("s") * 2 + lax.axis_index("c")
        row = table_v.shape[0] // num_heads
        q0s = wid * (q_per_worker * s_k)
        a0 = pl.multiple_of(
            lax.shift_left(lax.shift_right_logical(q0s, 7), 7), 128
        )
        shift = q0s - a0
        pltpu.sync_copy(table_hbm, table_v)
        pltpu.sync_copy(idx_hbm.at[pl.ds(a0, win)], idx_v)

        n_k = len(k_starts)
        k_last = k_starts[-1]

        def q_loop(qi, carry):
            @plsc.parallel_loop(0, n_k)
            def k_loop(ki):
                k0 = lax.min(ki * _LANES, k_last)
                iv = idx_v[pl.ds(shift + qi * s_k + k0, _LANES)]
                for hh in range(num_heads):
                    out_v[qi * num_heads + hh, pl.ds(k0, _LANES)] = (
                        plsc.load_gather(table_v, [iv + hh * row])
                    )

            return carry

        lax.fori_loop(0, q_per_worker, q_loop, 0)
        t0 = wid * (q_per_worker * num_heads)
        pltpu.sync_copy(
            out_v, out_hbm.at[pl.ds(t0, q_per_worker * num_heads), :]
        )

    return body


def _sc_gather(table_flat, idx_flat, num_heads, s_k, q_per_worker, k_starts, win):
    n_rows = _WORKERS * q_per_worker * num_heads
    mesh = plsc.VectorSubcoreMesh(core_axis_name="c", subcore_axis_name="s")
    return pl.kernel(
        _make_sc_body(num_heads, s_k, q_per_worker, k_starts, win),
        out_type=jax.ShapeDtypeStruct((n_rows, s_k), jnp.float32),
        mesh=mesh,
        compiler_params=pltpu.CompilerParams(
            needs_layout_passes=False, use_tc_tiling_on_sc=True
        ),
        scratch_types=[
            pltpu.VMEM((table_flat.shape[0],), jnp.float32),
            pltpu.VMEM((win,), jnp.int32),
            pltpu.VMEM((q_per_worker * num_heads, s_k), jnp.float32),
        ],
    )(table_flat, idx_flat)


def _add_body(x_ref, pos_ref, o_ref):
    s_q, h, s_k = x_ref.shape[1:]
    o_ref[...] = x_ref[...] + pos_ref[...].reshape(s_q, h, s_k)


def kernel(inputs, relative_position_bias_table, relative_position_index):
    b, h, s_q, s_k = inputs.shape

    n_tasks = s_q * h
    q_per_worker = -(-s_q // _WORKERS)

    k_starts = list(range(0, s_k - _LANES + 1, _LANES))
    if k_starts[-1] + _LANES < s_k:
        k_starts.append(s_k - _LANES)

    span = 128 + (q_per_worker - 1) * s_k + k_starts[-1] + _LANES
    win = ((span + 127) // 128) * 128
    a0_max = ((_WORKERS - 1) * q_per_worker * s_k) >> 7 << 7
    idx_len = a0_max + win

    idx = relative_position_index[:s_q, :s_k].astype(jnp.int32).reshape(-1)
    idx_pad = jnp.pad(idx, (0, idx_len - idx.size))
    nrd = relative_position_bias_table.shape[1]
    row = ((nrd + 127) // 128) * 128
    table_flat = jnp.pad(
        relative_position_bias_table, ((0, 0), (0, row - nrd))
    ).reshape(-1)

    pos2d = _sc_gather(
        table_flat, idx_pad, h, s_k, q_per_worker, k_starts, win
    )

    xt = jnp.transpose(inputs, (0, 2, 1, 3))
    out = pl.pallas_call(
        _add_body,
        out_shape=jax.ShapeDtypeStruct((b, s_q, h, s_k), jnp.float32),
        grid=(b // 4,),
        in_specs=[
            pl.BlockSpec((4, s_q, h, s_k), lambda i: (i, 0, 0, 0)),
            pl.BlockSpec((n_tasks, s_k), lambda i: (0, 0)),
        ],
        out_specs=pl.BlockSpec((4, s_q, h, s_k), lambda i: (i, 0, 0, 0)),
    )(xt, pos2d)
    return jnp.transpose(out, (0, 2, 1, 3))


# speedup vs baseline: 1.1040x; 1.0066x over previous
import jax
import jax.numpy as jnp
from jax import lax
from jax.experimental import pallas as pl
from jax.experimental.pallas import tpu as pltpu
from jax.experimental.pallas import tpu_sc as plsc

_LANES = 16
_WORKERS = 32


def _make_sc_body(num_heads, s_k, q_per_worker, k_starts):
    def body(table_hbm, idx_hbm, out_hbm, table_v, idx_v, out_v):
        wid = lax.axis_index("s") * 2 + lax.axis_index("c")
        q0 = wid * q_per_worker
        r0 = pl.multiple_of(
            lax.shift_left(lax.shift_right_logical(q0, 3), 3), 8
        )
        qoff = q0 - r0
        pltpu.sync_copy(table_hbm, table_v)
        pltpu.sync_copy(idx_hbm.at[pl.ds(r0, 2 * 8), :], idx_v)

        n_k = len(k_starts)
        k_last = k_starts[-1]
        hvecs = [jnp.full((_LANES,), hh, jnp.int32) for hh in range(num_heads)]

        def q_loop(qi, carry):
            @plsc.parallel_loop(0, n_k)
            def k_loop(ki):
                k0 = lax.min(ki * _LANES, k_last)
                iv = idx_v[qoff + qi, pl.ds(k0, _LANES)]
                for hh in range(num_heads):
                    out_v[qi * num_heads + hh, pl.ds(k0, _LANES)] = (
                        plsc.load_gather(table_v, [hvecs[hh], iv])
                    )

            return carry

        lax.fori_loop(0, q_per_worker, q_loop, 0)
        t0 = wid * (q_per_worker * num_heads)
        pltpu.sync_copy(
            out_v, out_hbm.at[pl.ds(t0, q_per_worker * num_heads), :]
        )

    return body


def _sc_gather(table, idx2, num_heads, s_k, q_per_worker, k_starts):
    n_rows = _WORKERS * q_per_worker * num_heads
    mesh = plsc.VectorSubcoreMesh(core_axis_name="c", subcore_axis_name="s")
    return pl.kernel(
        _make_sc_body(num_heads, s_k, q_per_worker, k_starts),
        out_type=jax.ShapeDtypeStruct((n_rows, s_k), jnp.float32),
        mesh=mesh,
        compiler_params=pltpu.CompilerParams(
            needs_layout_passes=False, use_tc_tiling_on_sc=True
        ),
        scratch_types=[
            pltpu.VMEM(table.shape, jnp.float32),
            pltpu.VMEM((2 * 8, s_k), jnp.int32),
            pltpu.VMEM((q_per_worker * num_heads, s_k), jnp.float32),
        ],
    )(table, idx2)


def _add_body(x_ref, pos_ref, o_ref):
    s_q, h, s_k = x_ref.shape[1:]
    o_ref[...] = x_ref[...] + pos_ref[...].reshape(s_q, h, s_k)


def kernel(inputs, relative_position_bias_table, relative_position_index):
    b, h, s_q, s_k = inputs.shape

    n_tasks = s_q * h
    q_per_worker = -(-s_q // _WORKERS)

    k_starts = list(range(0, s_k - _LANES + 1, _LANES))
    if k_starts[-1] + _LANES < s_k:
        k_starts.append(s_k - _LANES)

    pad_rows = ((_WORKERS - 1) * q_per_worker >> 3 << 3) + 2 * 8
    idx2 = jnp.pad(
        relative_position_index[:s_q, :s_k].astype(jnp.int32),
        ((0, pad_rows - s_q), (0, 0)),
    )

    pos2d = _sc_gather(
        relative_position_bias_table, idx2, h, s_k, q_per_worker, k_starts
    )

    xt = jnp.transpose(inputs, (0, 2, 1, 3))
    out = pl.pallas_call(
        _add_body,
        out_shape=jax.ShapeDtypeStruct((b, s_q, h, s_k), jnp.float32),
        grid=(b // 4,),
        in_specs=[
            pl.BlockSpec((4, s_q, h, s_k), lambda i: (i, 0, 0, 0)),
            pl.BlockSpec((n_tasks, s_k), lambda i: (0, 0)),
        ],
        out_specs=pl.BlockSpec((4, s_q, h, s_k), lambda i: (i, 0, 0, 0)),
    )(xt, pos2d)
    return jnp.transpose(out, (0, 2, 1, 3))
